# baseline (speedup 1.0000x reference)
import functools

import jax
import jax.numpy as jnp
from jax import lax
from jax.experimental import pallas as pl
from jax.experimental.pallas import tpu as pltpu
from jax.experimental.pallas import tpu_sc as plsc

NC = 2
NS = 16
NW = NC * NS
CH = 80
RN = 80
SUP = 25


def _rr_rows(s, n, copy_one):
    n_chunks = n // RN
    iters = (n_chunks + NS - 1) // NS

    def body(t, carry):
        cid = s + t * NS

        @pl.when(cid < n_chunks)
        def _():
            copy_one(cid * RN)

        return carry

    lax.fori_loop(0, iters, body, 0)


def _segsum_sc(x, src, dst, zeros):
    n, d = x.shape
    e = src.shape[0]
    e_per_w = e // NW
    n_chunks = e_per_w // CH
    mesh = plsc.VectorSubcoreMesh(core_axis_name="c", subcore_axis_name="s")

    @functools.partial(
        pl.kernel,
        out_type=jax.ShapeDtypeStruct((NC, n, d), jnp.float32),
        mesh=mesh,
        scratch_types=[
            pltpu.VMEM((SUP * CH,), jnp.int32),
            pltpu.VMEM((SUP * CH,), jnp.int32),
            pltpu.VMEM((CH, d), jnp.float32),
            pltpu.VMEM((CH, d), jnp.float32),
            pltpu.VMEM((CH, d), jnp.float32),
            pltpu.VMEM_SHARED((n, d), jnp.float32),
            pltpu.SemaphoreType.DMA,
            pltpu.SemaphoreType.DMA,
            pltpu.SemaphoreType.DMA,
            pltpu.SemaphoreType.DMA,
            pltpu.SemaphoreType.DMA,
            pltpu.SemaphoreType.DMA,
        ],
    )
    def k(x_hbm, src_hbm, dst_hbm, zeros_hbm, out_hbm,
          sidx, didx, rows0, rows1, rows2, acc, g0, g1, g2, s0, s1, s2):
        c = lax.axis_index("c")
        s = lax.axis_index("s")
        w = c * NS + s
        rows = (rows0, rows1, rows2)
        gsem = (g0, g1, g2)
        ssem = (s0, s1, s2)
        _rr_rows(s, n, lambda r0: pltpu.sync_copy(
            zeros_hbm.at[pl.ds(r0, RN)], acc.at[pl.ds(r0, RN)]))
        plsc.subcore_barrier()

        def start_g(j, k_):
            pltpu.async_copy(x_hbm.at[sidx.at[pl.ds(j * CH, CH)]],
                             rows[k_], gsem[k_])

        def wait_g(k_):
            pltpu.make_async_copy(zeros_hbm.at[pl.ds(0, CH)],
                                  rows[k_], gsem[k_]).wait()

        def start_s(j, k_):
            pltpu.async_copy(rows[k_], acc.at[didx.at[pl.ds(j * CH, CH)]],
                             ssem[k_], add=True)

        def wait_s(k_):
            pltpu.make_async_copy(zeros_hbm.at[pl.ds(0, CH)],
                                  rows[k_], ssem[k_]).wait()

        n_tri = SUP // 3
        tail = SUP - 3 * n_tri

        def super_body(u, carry):
            off = w * e_per_w + u * (SUP * CH)
            pltpu.sync_copy(src_hbm.at[pl.ds(off, SUP * CH)], sidx)
            pltpu.sync_copy(dst_hbm.at[pl.ds(off, SUP * CH)], didx)
            for k_ in range(min(3, SUP)):
                start_g(k_, k_)

            def body(t, carry2):
                a = 3 * t
                for k_ in range(3):
                    wait_g(k_)
                    start_s(a + k_, k_)
                for k_ in range(3):

                    @pl.when(a + k_ + 3 < SUP)
                    def _(k_=k_):
                        wait_s(k_)
                        start_g(a + k_ + 3, k_)

                return carry2

            lax.fori_loop(0, n_tri, body, 0)
            for k_ in range(tail):
                wait_g(k_)
                start_s(3 * n_tri + k_, k_)
            for k_ in range(min(3, SUP)):
                wait_s(k_)
            return carry

        lax.fori_loop(0, n_chunks // SUP, super_body, 0)
        plsc.subcore_barrier()
        _rr_rows(s, n, lambda r0: pltpu.sync_copy(
            acc.at[pl.ds(r0, RN)], out_hbm.at[c, pl.ds(r0, RN)]))

    return k(x, src, dst, zeros)


def _count_sc(dst, zeros, n, d):
    e = dst.shape[0]
    e_per_w = e // NW
    n_chunks = e_per_w // CH
    depth = 4
    mesh = plsc.VectorSubcoreMesh(core_axis_name="c", subcore_axis_name="s")

    @functools.partial(
        pl.kernel,
        out_type=jax.ShapeDtypeStruct((NC, n, d), jnp.float32),
        mesh=mesh,
        scratch_types=[
            pltpu.VMEM((SUP * CH,), jnp.int32),
            pltpu.VMEM((CH, d), jnp.float32),
            pltpu.VMEM_SHARED((n, d), jnp.float32),
            pltpu.SemaphoreType.DMA,
        ],
    )
    def k(dst_hbm, zeros_hbm, out_hbm, didx, ones_v, acc, ssem):
        c = lax.axis_index("c")
        s = lax.axis_index("s")
        w = c * NS + s

        def fill(i, carry):
            for kk in range(d // 16):
                ones_v[i, pl.ds(kk * 16, 16)] = jnp.ones((16,), jnp.float32)
            return carry

        lax.fori_loop(0, CH, fill, 0)
        _rr_rows(s, n, lambda r0: pltpu.sync_copy(
            zeros_hbm.at[pl.ds(r0, RN)], acc.at[pl.ds(r0, RN)]))
        plsc.subcore_barrier()

        def wait_one():
            pltpu.make_async_copy(zeros_hbm.at[pl.ds(0, CH)], ones_v, ssem).wait()

        def super_body(u, carry):
            off = w * e_per_w + u * (SUP * CH)
            pltpu.sync_copy(dst_hbm.at[pl.ds(off, SUP * CH)], didx)

            def body(j, carry2):
                pltpu.async_copy(ones_v, acc.at[didx.at[pl.ds(j * CH, CH)]],
                                 ssem, add=True)

                @pl.when(j >= depth)
                def _():
                    wait_one()

                return carry2

            lax.fori_loop(0, SUP, body, 0)
            for _ in range(min(depth, SUP)):
                wait_one()
            return carry

        lax.fori_loop(0, n_chunks // SUP, super_body, 0)
        plsc.subcore_barrier()
        _rr_rows(s, n, lambda r0: pltpu.sync_copy(
            acc.at[pl.ds(r0, RN)], out_hbm.at[c, pl.ds(r0, RN)]))

    return k(dst, zeros)


def _dense_tc(aggp, cntp, h, wl_t, bl, wr_t, relu):
    n, d = h.shape
    bn = 1000

    def body(ap_ref, cp_ref, h_ref, wl_ref, bl_ref, wr_ref, o_ref):
        agg = ap_ref[0] + ap_ref[1]
        cnt = cp_ref[0, :, 0:1] + cp_ref[1, :, 0:1]
        mean = agg / jnp.maximum(cnt, 1.0)
        y = (jnp.dot(mean, wl_ref[...], preferred_element_type=jnp.float32,
                     precision=lax.Precision.HIGHEST)
             + bl_ref[...]
             + jnp.dot(h_ref[...], wr_ref[...], preferred_element_type=jnp.float32,
                       precision=lax.Precision.HIGHEST))
        if relu:
            y = jnp.maximum(y, 0.0)
        o_ref[...] = y

    return pl.pallas_call(
        body,
        out_shape=jax.ShapeDtypeStruct((n, d), jnp.float32),
        grid=(n // bn,),
        in_specs=[
            pl.BlockSpec((NC, bn, d), lambda i: (0, i, 0)),
            pl.BlockSpec((NC, bn, d), lambda i: (0, i, 0)),
            pl.BlockSpec((bn, d), lambda i: (i, 0)),
            pl.BlockSpec((d, d), lambda i: (0, 0)),
            pl.BlockSpec((1, d), lambda i: (0, 0)),
            pl.BlockSpec((d, d), lambda i: (0, 0)),
        ],
        out_specs=pl.BlockSpec((bn, d), lambda i: (i, 0)),
    )(aggp, cntp, h, wl_t, bl, wr_t)


def kernel(x, edge_index, W1l, b1, W1r, W2l, b2, W2r, W3l, b3, W3r):
    n, d = x.shape
    ei = edge_index.astype(jnp.int32)
    src, dst = ei[0], ei[1]
    zeros = jnp.zeros((n, d), jnp.float32)

    cntp = _count_sc(dst, zeros, n, d)

    h = x
    for wl, bl, wr, relu in (
        (W1l, b1, W1r, True),
        (W2l, b2, W2r, True),
        (W3l, b3, W3r, False),
    ):
        aggp = _segsum_sc(h, src, dst, zeros)
        h = _dense_tc(aggp, cntp, h, wl.T, bl.reshape(1, d), wr.T, relu)
    return h


# speedup vs baseline: 1.1482x; 1.1482x over previous
import functools

import jax
import jax.numpy as jnp
from jax import lax
from jax.experimental import pallas as pl
from jax.experimental.pallas import tpu as pltpu
from jax.experimental.pallas import tpu_sc as plsc

NC = 2
NS = 16
NW = NC * NS
CH = 80
RN = 80
SUP = 25


def _rr_rows(s, n, copy_one):
    n_chunks = n // RN
    iters = (n_chunks + NS - 1) // NS

    def body(t, carry):
        cid = s + t * NS

        @pl.when(cid < n_chunks)
        def _():
            copy_one(cid * RN)

        return carry

    lax.fori_loop(0, iters, body, 0)


def _segsum_sc(x, src, dst, zeros):
    n, d = x.shape
    e = src.shape[0]
    e_per_w = e // NW
    n_chunks = e_per_w // CH
    mesh = plsc.VectorSubcoreMesh(core_axis_name="c", subcore_axis_name="s")

    @functools.partial(
        pl.kernel,
        out_type=jax.ShapeDtypeStruct((NC, n, d), jnp.float32),
        mesh=mesh,
        scratch_types=[
            pltpu.VMEM((SUP * CH,), jnp.int32),
            pltpu.VMEM((SUP * CH,), jnp.int32),
            pltpu.VMEM((CH, d), jnp.float32),
            pltpu.VMEM((CH, d), jnp.float32),
            pltpu.VMEM((CH, d), jnp.float32),
            pltpu.VMEM_SHARED((n, d), jnp.float32),
            pltpu.SemaphoreType.DMA,
            pltpu.SemaphoreType.DMA,
            pltpu.SemaphoreType.DMA,
            pltpu.SemaphoreType.DMA,
            pltpu.SemaphoreType.DMA,
            pltpu.SemaphoreType.DMA,
        ],
    )
    def k(x_hbm, src_hbm, dst_hbm, zeros_hbm, out_hbm,
          sidx, didx, rows0, rows1, rows2, acc, g0, g1, g2, s0, s1, s2):
        c = lax.axis_index("c")
        s = lax.axis_index("s")
        w = c * NS + s
        rows = (rows0, rows1, rows2)
        gsem = (g0, g1, g2)
        ssem = (s0, s1, s2)
        _rr_rows(s, n, lambda r0: pltpu.sync_copy(
            zeros_hbm.at[pl.ds(r0, RN)], acc.at[pl.ds(r0, RN)]))
        plsc.subcore_barrier()

        def start_g(j, k_):
            pltpu.async_copy(x_hbm.at[sidx.at[pl.ds(j * CH, CH)]],
                             rows[k_], gsem[k_])

        def wait_g(k_):
            pltpu.make_async_copy(zeros_hbm.at[pl.ds(0, CH)],
                                  rows[k_], gsem[k_]).wait()

        def start_s(j, k_):
            pltpu.async_copy(rows[k_], acc.at[didx.at[pl.ds(j * CH, CH)]],
                             ssem[k_], add=True)

        def wait_s(k_):
            pltpu.make_async_copy(zeros_hbm.at[pl.ds(0, CH)],
                                  rows[k_], ssem[k_]).wait()

        n_tri = SUP // 3
        tail = SUP - 3 * n_tri

        def super_body(u, carry):
            off = w * e_per_w + u * (SUP * CH)
            pltpu.sync_copy(src_hbm.at[pl.ds(off, SUP * CH)], sidx)
            pltpu.sync_copy(dst_hbm.at[pl.ds(off, SUP * CH)], didx)
            for k_ in range(min(3, SUP)):
                start_g(k_, k_)

            def body(t, carry2):
                a = 3 * t
                for k_ in range(3):
                    wait_g(k_)
                    pltpu.sync_copy(rows[k_],
                                    acc.at[didx.at[pl.ds((a + k_) * CH, CH)]],
                                    add=True)

                    @pl.when(a + k_ + 3 < SUP)
                    def _(k_=k_):
                        start_g(a + k_ + 3, k_)

                return carry2

            lax.fori_loop(0, n_tri, body, 0)
            for k_ in range(tail):
                wait_g(k_)
                pltpu.sync_copy(rows[k_],
                                acc.at[didx.at[pl.ds((3 * n_tri + k_) * CH, CH)]],
                                add=True)
            return carry

        lax.fori_loop(0, n_chunks // SUP, super_body, 0)
        plsc.subcore_barrier()
        _rr_rows(s, n, lambda r0: pltpu.sync_copy(
            acc.at[pl.ds(r0, RN)], out_hbm.at[c, pl.ds(r0, RN)]))

    return k(x, src, dst, zeros)


def _count_sc(dst, zeros, n, d):
    e = dst.shape[0]
    e_per_w = e // NW
    n_chunks = e_per_w // CH
    depth = 4
    mesh = plsc.VectorSubcoreMesh(core_axis_name="c", subcore_axis_name="s")

    @functools.partial(
        pl.kernel,
        out_type=jax.ShapeDtypeStruct((NC, n, d), jnp.float32),
        mesh=mesh,
        scratch_types=[
            pltpu.VMEM((SUP * CH,), jnp.int32),
            pltpu.VMEM((CH, d), jnp.float32),
            pltpu.VMEM_SHARED((n, d), jnp.float32),
            pltpu.SemaphoreType.DMA,
        ],
    )
    def k(dst_hbm, zeros_hbm, out_hbm, didx, ones_v, acc, ssem):
        c = lax.axis_index("c")
        s = lax.axis_index("s")
        w = c * NS + s

        def fill(i, carry):
            for kk in range(d // 16):
                ones_v[i, pl.ds(kk * 16, 16)] = jnp.ones((16,), jnp.float32)
            return carry

        lax.fori_loop(0, CH, fill, 0)
        _rr_rows(s, n, lambda r0: pltpu.sync_copy(
            zeros_hbm.at[pl.ds(r0, RN)], acc.at[pl.ds(r0, RN)]))
        plsc.subcore_barrier()

        def wait_one():
            pltpu.make_async_copy(zeros_hbm.at[pl.ds(0, CH)], ones_v, ssem).wait()

        def super_body(u, carry):
            off = w * e_per_w + u * (SUP * CH)
            pltpu.sync_copy(dst_hbm.at[pl.ds(off, SUP * CH)], didx)

            def body(j, carry2):
                pltpu.async_copy(ones_v, acc.at[didx.at[pl.ds(j * CH, CH)]],
                                 ssem, add=True)

                @pl.when(j >= depth)
                def _():
                    wait_one()

                return carry2

            lax.fori_loop(0, SUP, body, 0)
            for _ in range(min(depth, SUP)):
                wait_one()
            return carry

        lax.fori_loop(0, n_chunks // SUP, super_body, 0)
        plsc.subcore_barrier()
        _rr_rows(s, n, lambda r0: pltpu.sync_copy(
            acc.at[pl.ds(r0, RN)], out_hbm.at[c, pl.ds(r0, RN)]))

    return k(dst, zeros)


def _dense_tc(aggp, cntp, h, wl_t, bl, wr_t, relu):
    n, d = h.shape
    bn = 1000

    def body(ap_ref, cp_ref, h_ref, wl_ref, bl_ref, wr_ref, o_ref):
        agg = ap_ref[0] + ap_ref[1]
        cnt = cp_ref[0, :, 0:1] + cp_ref[1, :, 0:1]
        mean = agg / jnp.maximum(cnt, 1.0)
        y = (jnp.dot(mean, wl_ref[...], preferred_element_type=jnp.float32,
                     precision=lax.Precision.HIGHEST)
             + bl_ref[...]
             + jnp.dot(h_ref[...], wr_ref[...], preferred_element_type=jnp.float32,
                       precision=lax.Precision.HIGHEST))
        if relu:
            y = jnp.maximum(y, 0.0)
        o_ref[...] = y

    return pl.pallas_call(
        body,
        out_shape=jax.ShapeDtypeStruct((n, d), jnp.float32),
        grid=(n // bn,),
        in_specs=[
            pl.BlockSpec((NC, bn, d), lambda i: (0, i, 0)),
            pl.BlockSpec((NC, bn, d), lambda i: (0, i, 0)),
            pl.BlockSpec((bn, d), lambda i: (i, 0)),
            pl.BlockSpec((d, d), lambda i: (0, 0)),
            pl.BlockSpec((1, d), lambda i: (0, 0)),
            pl.BlockSpec((d, d), lambda i: (0, 0)),
        ],
        out_specs=pl.BlockSpec((bn, d), lambda i: (i, 0)),
    )(aggp, cntp, h, wl_t, bl, wr_t)


def kernel(x, edge_index, W1l, b1, W1r, W2l, b2, W2r, W3l, b3, W3r):
    n, d = x.shape
    ei = edge_index.astype(jnp.int32)
    src, dst = ei[0], ei[1]
    zeros = jnp.zeros((n, d), jnp.float32)

    cntp = _count_sc(dst, zeros, n, d)

    h = x
    for wl, bl, wr, relu in (
        (W1l, b1, W1r, True),
        (W2l, b2, W2r, True),
        (W3l, b3, W3r, False),
    ):
        aggp = _segsum_sc(h, src, dst, zeros)
        h = _dense_tc(aggp, cntp, h, wl.T, bl.reshape(1, d), wr.T, relu)
    return h


# speedup vs baseline: 1.1736x; 1.0221x over previous
import functools

import jax
import jax.numpy as jnp
from jax import lax
from jax.experimental import pallas as pl
from jax.experimental.pallas import tpu as pltpu
from jax.experimental.pallas import tpu_sc as plsc

NC = 2
NS = 16
NW = NC * NS
CH = 80
RN = 80
SUP = 25


def _rr_rows(s, n, copy_one):
    n_chunks = n // RN
    iters = (n_chunks + NS - 1) // NS

    def body(t, carry):
        cid = s + t * NS

        @pl.when(cid < n_chunks)
        def _():
            copy_one(cid * RN)

        return carry

    lax.fori_loop(0, iters, body, 0)


def _segsum_sc(x, src, dst, zeros):
    n, d = x.shape
    e = src.shape[0]
    e_per_w = e // NW
    n_chunks = e_per_w // CH
    mesh = plsc.VectorSubcoreMesh(core_axis_name="c", subcore_axis_name="s")

    @functools.partial(
        pl.kernel,
        out_type=jax.ShapeDtypeStruct((NC, n, d), jnp.float32),
        mesh=mesh,
        scratch_types=[
            pltpu.VMEM((SUP * CH,), jnp.int32),
            pltpu.VMEM((SUP * CH,), jnp.int32),
            pltpu.VMEM((CH, d), jnp.float32),
            pltpu.VMEM((CH, d), jnp.float32),
            pltpu.VMEM((CH, d), jnp.float32),
            pltpu.VMEM_SHARED((n, d), jnp.float32),
            pltpu.SemaphoreType.DMA,
            pltpu.SemaphoreType.DMA,
            pltpu.SemaphoreType.DMA,
            pltpu.SemaphoreType.DMA,
            pltpu.SemaphoreType.DMA,
            pltpu.SemaphoreType.DMA,
        ],
    )
    def k(x_hbm, src_hbm, dst_hbm, zeros_hbm, out_hbm,
          sidx, didx, rows0, rows1, rows2, acc, g0, g1, g2, s0, s1, s2):
        c = lax.axis_index("c")
        s = lax.axis_index("s")
        w = c * NS + s
        rows = (rows0, rows1, rows2)
        gsem = (g0, g1, g2)
        ssem = (s0, s1, s2)
        _rr_rows(s, n, lambda r0: pltpu.sync_copy(
            zeros_hbm.at[pl.ds(r0, RN)], acc.at[pl.ds(r0, RN)]))
        plsc.subcore_barrier()

        def start_g(j, k_):
            pltpu.async_copy(x_hbm.at[sidx.at[pl.ds(j * CH, CH)]],
                             rows[k_], gsem[k_])

        def wait_g(k_):
            pltpu.make_async_copy(zeros_hbm.at[pl.ds(0, CH)],
                                  rows[k_], gsem[k_]).wait()

        def start_s(j, k_):
            pltpu.async_copy(rows[k_], acc.at[didx.at[pl.ds(j * CH, CH)]],
                             ssem[k_], add=True)

        def wait_s(k_):
            pltpu.make_async_copy(zeros_hbm.at[pl.ds(0, CH)],
                                  rows[k_], ssem[k_]).wait()

        n_tri = SUP // 3
        tail = SUP - 3 * n_tri

        def super_body(u, carry):
            off = w * e_per_w + u * (SUP * CH)
            pltpu.sync_copy(src_hbm.at[pl.ds(off, SUP * CH)], sidx)
            pltpu.sync_copy(dst_hbm.at[pl.ds(off, SUP * CH)], didx)
            for k_ in range(min(3, SUP)):
                start_g(k_, k_)

            def body(t, carry2):
                a = 3 * t
                for k_ in range(3):
                    wait_g(k_)
                    pltpu.sync_copy(rows[k_],
                                    acc.at[didx.at[pl.ds((a + k_) * CH, CH)]],
                                    add=True)

                    @pl.when(a + k_ + 3 < SUP)
                    def _(k_=k_):
                        start_g(a + k_ + 3, k_)

                return carry2

            lax.fori_loop(0, n_tri, body, 0)
            for k_ in range(tail):
                wait_g(k_)
                pltpu.sync_copy(rows[k_],
                                acc.at[didx.at[pl.ds((3 * n_tri + k_) * CH, CH)]],
                                add=True)
            return carry

        lax.fori_loop(0, n_chunks // SUP, super_body, 0)
        plsc.subcore_barrier()
        _rr_rows(s, n, lambda r0: pltpu.sync_copy(
            acc.at[pl.ds(r0, RN)], out_hbm.at[c, pl.ds(r0, RN)]))

    return k(x, src, dst, zeros)


def _count_sc(dst, zeros, n, d):
    e = dst.shape[0]
    e_per_w = e // NW
    n_chunks = e_per_w // CH
    depth = 4
    mesh = plsc.VectorSubcoreMesh(core_axis_name="c", subcore_axis_name="s")

    @functools.partial(
        pl.kernel,
        out_type=jax.ShapeDtypeStruct((NC, n, d), jnp.float32),
        mesh=mesh,
        scratch_types=[
            pltpu.VMEM((SUP * CH,), jnp.int32),
            pltpu.VMEM((CH, d), jnp.float32),
            pltpu.VMEM_SHARED((n, d), jnp.float32),
            pltpu.SemaphoreType.DMA,
        ],
    )
    def k(dst_hbm, zeros_hbm, out_hbm, didx, ones_v, acc, ssem):
        c = lax.axis_index("c")
        s = lax.axis_index("s")
        w = c * NS + s

        def fill(i, carry):
            for kk in range(d // 16):
                ones_v[i, pl.ds(kk * 16, 16)] = jnp.ones((16,), jnp.float32)
            return carry

        lax.fori_loop(0, CH, fill, 0)
        _rr_rows(s, n, lambda r0: pltpu.sync_copy(
            zeros_hbm.at[pl.ds(r0, RN)], acc.at[pl.ds(r0, RN)]))
        plsc.subcore_barrier()

        def wait_one():
            pltpu.make_async_copy(zeros_hbm.at[pl.ds(0, CH)], ones_v, ssem).wait()

        def super_body(u, carry):
            off = w * e_per_w + u * (SUP * CH)
            pltpu.sync_copy(dst_hbm.at[pl.ds(off, SUP * CH)], didx)

            def body(j, carry2):
                pltpu.async_copy(ones_v, acc.at[didx.at[pl.ds(j * CH, CH)]],
                                 ssem, add=True)

                @pl.when(j >= depth)
                def _():
                    wait_one()

                return carry2

            lax.fori_loop(0, SUP, body, 0)
            for _ in range(min(depth, SUP)):
                wait_one()
            return carry

        lax.fori_loop(0, n_chunks // SUP, super_body, 0)
        plsc.subcore_barrier()
        _rr_rows(s, n, lambda r0: pltpu.sync_copy(
            acc.at[pl.ds(r0, RN)], out_hbm.at[c, pl.ds(r0, RN)]))

    return k(dst, zeros)


def _self_tc(h, wr_t, bl):
    n, d = h.shape
    bn = 1000

    def body(h_ref, wr_ref, bl_ref, o_ref):
        o_ref[...] = (jnp.dot(h_ref[...], wr_ref[...],
                              preferred_element_type=jnp.float32,
                              precision=lax.Precision.HIGHEST)
                      + bl_ref[...])

    return pl.pallas_call(
        body,
        out_shape=jax.ShapeDtypeStruct((n, d), jnp.float32),
        grid=(n // bn,),
        in_specs=[
            pl.BlockSpec((bn, d), lambda i: (i, 0)),
            pl.BlockSpec((d, d), lambda i: (0, 0)),
            pl.BlockSpec((1, d), lambda i: (0, 0)),
        ],
        out_specs=pl.BlockSpec((bn, d), lambda i: (i, 0)),
    )(h, wr_t, bl)


def _combine_tc(aggp, cntp, selfv, wl_t, relu):
    n, d = selfv.shape
    bn = 1000

    def body(ap_ref, cp_ref, sv_ref, wl_ref, o_ref):
        agg = ap_ref[0] + ap_ref[1]
        cnt = cp_ref[0, :, 0:1] + cp_ref[1, :, 0:1]
        mean = agg / jnp.maximum(cnt, 1.0)
        y = (jnp.dot(mean, wl_ref[...], preferred_element_type=jnp.float32,
                     precision=lax.Precision.HIGHEST)
             + sv_ref[...])
        if relu:
            y = jnp.maximum(y, 0.0)
        o_ref[...] = y

    return pl.pallas_call(
        body,
        out_shape=jax.ShapeDtypeStruct((n, d), jnp.float32),
        grid=(n // bn,),
        in_specs=[
            pl.BlockSpec((NC, bn, d), lambda i: (0, i, 0)),
            pl.BlockSpec((NC, bn, d), lambda i: (0, i, 0)),
            pl.BlockSpec((bn, d), lambda i: (i, 0)),
            pl.BlockSpec((d, d), lambda i: (0, 0)),
        ],
        out_specs=pl.BlockSpec((bn, d), lambda i: (i, 0)),
    )(aggp, cntp, selfv, wl_t)


def kernel(x, edge_index, W1l, b1, W1r, W2l, b2, W2r, W3l, b3, W3r):
    n, d = x.shape
    ei = edge_index.astype(jnp.int32)
    src, dst = ei[0], ei[1]
    zeros = jnp.zeros((n, d), jnp.float32)

    cntp = _count_sc(dst, zeros, n, d)

    h = x
    for wl, bl, wr, relu in (
        (W1l, b1, W1r, True),
        (W2l, b2, W2r, True),
        (W3l, b3, W3r, False),
    ):
        selfv = _self_tc(h, wr.T, bl.reshape(1, d))
        aggp = _segsum_sc(h, src, dst, zeros)
        h = _combine_tc(aggp, cntp, selfv, wl.T, relu)
    return h
